# baseline (speedup 1.0000x reference)
import functools

import jax
import jax.numpy as jnp
from jax import lax
from jax.experimental import pallas as pl
from jax.experimental.pallas import tpu as pltpu
from jax.experimental.pallas import tpu_sc as plsc

B = 1024
L = 200
D = 64
VOCAB = 1000000
ROWS = B * L
NC = 2
NS = 16
NW = NC * NS
BPW = B // NW
EPS = 1e-5
LN_B = 64
PACK = LN_B * L // 2


def _ptab_body(pos_ref, typ_ref, out_ref):
    p = pos_ref[...]
    out_ref[0] = p + typ_ref[0:1, :]
    out_ref[1] = p + typ_ref[1:2, :]


def _build_ptab(pos_table, type_table):
    out = pl.pallas_call(
        _ptab_body,
        out_shape=jax.ShapeDtypeStruct((2, L, D), jnp.float32),
    )(pos_table, type_table)
    return out.reshape(2 * L, D)


RELAY_CH = 24576


def _relayout_body(t_ref, out_ref):
    tr = t_ref[...].T
    out_ref[...] = jnp.concatenate([tr, tr], axis=1)


def _relayout_table(tok_t):
    return pl.pallas_call(
        _relayout_body,
        grid=(pl.cdiv(VOCAB, RELAY_CH),),
        in_specs=[pl.BlockSpec((D, RELAY_CH), lambda i: (0, i))],
        out_specs=pl.BlockSpec((RELAY_CH, 2 * D), lambda i: (i, 0)),
        out_shape=jax.ShapeDtypeStruct((VOCAB, 2 * D), jnp.float32),
    )(tok_t)


def _sc_body(tok_hbm, ptab_hbm, tidx_hbm, pidx_hbm, sidx_hbm, out_hbm,
             tidx_v, pidx_v, sidx_v, cidx_v, trow0_v, prow0_v, trow1_v,
             prow1_v, sem_t0, sem_p0, sem_t1, sem_p1):
    wid = lax.axis_index("s") * NC + lax.axis_index("c")
    b0 = wid * BPW

    pltpu.sync_copy(tidx_hbm.at[pl.ds(b0, BPW)], tidx_v)
    pltpu.sync_copy(pidx_hbm.at[pl.ds(b0, BPW)], pidx_v)
    pltpu.sync_copy(sidx_hbm.at[pl.ds(b0, BPW)], sidx_v)

    def _cidx_body(b, _):
        for j in range(L // 16):
            sl = pl.ds(j * 16, 16)
            cidx_v[b, sl] = sidx_v[b, sl] * L + pidx_v[b, sl]
        sl = pl.ds(L - 16, 16)
        cidx_v[b, sl] = sidx_v[b, sl] * L + pidx_v[b, sl]
        return 0

    lax.fori_loop(0, BPW, _cidx_body, 0)

    def _start(b, trow, prow, sem_t, sem_p):
        pltpu.async_copy(tok_hbm.at[tidx_v.at[b]], trow, sem_t)
        pltpu.async_copy(ptab_hbm.at[cidx_v.at[b]], prow, sem_p)

    def _finish(b, trow, prow, sem_t, sem_p):
        pltpu.make_async_copy(tok_hbm.at[tidx_v.at[b]], trow, sem_t).wait()
        pltpu.make_async_copy(ptab_hbm.at[cidx_v.at[b]], prow, sem_p).wait()

        def _row_body(i, _):
            for u in range(2):
                r = 2 * i + u
                for q in range(D // 16):
                    sl = pl.ds(q * 16, 16)
                    trow[r, sl] = trow[r, sl] + prow[r, sl]
            return 0

        lax.fori_loop(0, L // 2, _row_body, 0)
        r0 = (b0 + b) * (L // 2)
        pltpu.sync_copy(trow.at[pl.ds(0, L // 2), pl.ds(0, D)],
                        out_hbm.at[pl.ds(r0, L // 2), pl.ds(0, D)])
        pltpu.sync_copy(trow.at[pl.ds(L // 2, L // 2), pl.ds(0, D)],
                        out_hbm.at[pl.ds(r0, L // 2), pl.ds(D, D)])

    _start(0, trow0_v, prow0_v, sem_t0, sem_p0)

    def _pipe_body(g, _):
        _start(2 * g + 1, trow1_v, prow1_v, sem_t1, sem_p1)
        _finish(2 * g, trow0_v, prow0_v, sem_t0, sem_p0)

        @pl.when(g < BPW // 2 - 1)
        def _():
            _start(2 * g + 2, trow0_v, prow0_v, sem_t0, sem_p0)

        _finish(2 * g + 1, trow1_v, prow1_v, sem_t1, sem_p1)
        return 0

    lax.fori_loop(0, BPW // 2, _pipe_body, 0)


def _sc_gather_sum(tok_table, ptab, tidx, pidx, sidx):
    mesh = plsc.VectorSubcoreMesh(core_axis_name="c", subcore_axis_name="s")
    fn = functools.partial(
        pl.kernel,
        mesh=mesh,
        compiler_params=pltpu.CompilerParams(use_tc_tiling_on_sc=False),
        out_type=jax.ShapeDtypeStruct((ROWS // 2, 2 * D), jnp.float32),
        scratch_types=[
            pltpu.VMEM((BPW, L), jnp.int32),
            pltpu.VMEM((BPW, L), jnp.int32),
            pltpu.VMEM((BPW, L), jnp.int32),
            pltpu.VMEM((BPW, L), jnp.int32),
            pltpu.VMEM((L, 2 * D), jnp.float32),
            pltpu.VMEM((L, D), jnp.float32),
            pltpu.VMEM((L, 2 * D), jnp.float32),
            pltpu.VMEM((L, D), jnp.float32),
            pltpu.SemaphoreType.DMA,
            pltpu.SemaphoreType.DMA,
            pltpu.SemaphoreType.DMA,
            pltpu.SemaphoreType.DMA,
        ],
    )(_sc_body)
    return fn(tok_table, ptab, tidx, pidx, sidx)


def _mask_body(m_ref, out_ref):
    out_ref[...] = jnp.broadcast_to(m_ref[...], (8, D, B))


def _build_mask(mask_row):
    out = pl.pallas_call(
        _mask_body,
        grid=(L // 8,),
        in_specs=[pl.BlockSpec((1, D, 1), lambda i: (0, 0, 0))],
        out_specs=pl.BlockSpec((8, D, B), lambda i: (i, 0, 0)),
        out_shape=jax.ShapeDtypeStruct((L, D, B), jnp.float32),
    )(mask_row)
    return out.transpose(2, 0, 1)


def _ln_body(x_ref, g_ref, b_ref, y_ref):
    x = x_ref[...].astype(jnp.float32)
    ii = lax.broadcasted_iota(jnp.int32, (2 * D, 2 * D), 0) // D
    jj = lax.broadcasted_iota(jnp.int32, (2 * D, 2 * D), 1) // D
    avg = jnp.where(ii == jj, 1.0 / D, 0.0).astype(jnp.float32)
    m = jnp.dot(x, avg, preferred_element_type=jnp.float32)
    s2 = jnp.dot(x * x, avg, preferred_element_type=jnp.float32)
    inv = lax.rsqrt(s2 - m * m + EPS)
    y = (x - m) * inv * g_ref[...] + b_ref[...]
    ya = y[:, :D].reshape(LN_B, L // 2, D)
    yb = y[:, D:].reshape(LN_B, L // 2, D)
    y_ref[...] = jnp.concatenate([ya, yb], axis=1)


def _ln_norm(packed, gamma2, beta2):
    grid = (B // LN_B,)
    return pl.pallas_call(
        _ln_body,
        grid=grid,
        in_specs=[
            pl.BlockSpec((PACK, 2 * D), lambda i: (i, 0)),
            pl.BlockSpec((1, 2 * D), lambda i: (0, 0)),
            pl.BlockSpec((1, 2 * D), lambda i: (0, 0)),
        ],
        out_specs=pl.BlockSpec((LN_B, L, D), lambda i: (i, 0, 0)),
        out_shape=jax.ShapeDtypeStruct((B, L, D), jnp.float32),
    )(packed, gamma2, beta2)


def kernel(input_token, position_ids, segment_ids, tok_table, type_table,
           pos_table, ln_gamma, ln_beta):
    ptab = _build_ptab(pos_table, type_table)
    tokp = _relayout_table(tok_table.T)
    packed = _sc_gather_sum(tokp, ptab, input_token, position_ids,
                            segment_ids)

    gamma2 = jnp.concatenate([ln_gamma, ln_gamma]).reshape(1, 2 * D)
    beta2 = jnp.concatenate([ln_beta, ln_beta]).reshape(1, 2 * D)
    mask_row = lax.slice(tok_table, (103, 0), (104, D))
    mask = _build_mask(mask_row.reshape(1, D, 1))
    y = _ln_norm(packed, gamma2, beta2)
    return y, mask


# speedup vs baseline: 1.0025x; 1.0025x over previous
import functools

import jax
import jax.numpy as jnp
from jax import lax
from jax.experimental import pallas as pl
from jax.experimental.pallas import tpu as pltpu
from jax.experimental.pallas import tpu_sc as plsc

B = 1024
L = 200
D = 64
VOCAB = 1000000
ROWS = B * L
NC = 2
NS = 16
NW = NC * NS
BPW = B // NW
EPS = 1e-5
LN_B = 64
PACK = LN_B * L // 2


def _ptab_body(pos_ref, typ_ref, out_ref):
    p = pos_ref[...]
    out_ref[0] = p + typ_ref[0:1, :]
    out_ref[1] = p + typ_ref[1:2, :]


def _build_ptab(pos_table, type_table):
    out = pl.pallas_call(
        _ptab_body,
        out_shape=jax.ShapeDtypeStruct((2, L, D), jnp.float32),
    )(pos_table, type_table)
    return out.reshape(2 * L, D)


RELAY_CH = 28672


def _relayout_body(t_ref, out_ref):
    tr = t_ref[...].T
    out_ref[...] = jnp.concatenate([tr, tr], axis=1)


def _relayout_table(tok_t):
    return pl.pallas_call(
        _relayout_body,
        grid=(pl.cdiv(VOCAB, RELAY_CH),),
        in_specs=[pl.BlockSpec((D, RELAY_CH), lambda i: (0, i))],
        out_specs=pl.BlockSpec((RELAY_CH, 2 * D), lambda i: (i, 0)),
        out_shape=jax.ShapeDtypeStruct((VOCAB, 2 * D), jnp.float32),
    )(tok_t)


def _sc_body(tok_hbm, ptab_hbm, tidx_hbm, pidx_hbm, sidx_hbm, out_hbm,
             tidx_v, pidx_v, sidx_v, cidx_v, trow0_v, prow0_v, trow1_v,
             prow1_v, sem_t0, sem_p0, sem_t1, sem_p1):
    wid = lax.axis_index("s") * NC + lax.axis_index("c")
    b0 = wid * BPW

    pltpu.sync_copy(tidx_hbm.at[pl.ds(b0, BPW)], tidx_v)
    pltpu.sync_copy(pidx_hbm.at[pl.ds(b0, BPW)], pidx_v)
    pltpu.sync_copy(sidx_hbm.at[pl.ds(b0, BPW)], sidx_v)

    def _cidx_body(b, _):
        for j in range(L // 16):
            sl = pl.ds(j * 16, 16)
            cidx_v[b, sl] = sidx_v[b, sl] * L + pidx_v[b, sl]
        sl = pl.ds(L - 16, 16)
        cidx_v[b, sl] = sidx_v[b, sl] * L + pidx_v[b, sl]
        return 0

    lax.fori_loop(0, BPW, _cidx_body, 0)

    def _start(b, trow, prow, sem_t, sem_p):
        pltpu.async_copy(tok_hbm.at[tidx_v.at[b]], trow, sem_t)
        pltpu.async_copy(ptab_hbm.at[cidx_v.at[b]], prow, sem_p)

    def _finish(b, trow, prow, sem_t, sem_p):
        pltpu.make_async_copy(tok_hbm.at[tidx_v.at[b]], trow, sem_t).wait()
        pltpu.make_async_copy(ptab_hbm.at[cidx_v.at[b]], prow, sem_p).wait()

        def _row_body(i, _):
            for u in range(2):
                r = 2 * i + u
                for q in range(D // 16):
                    sl = pl.ds(q * 16, 16)
                    trow[r, sl] = trow[r, sl] + prow[r, sl]
            return 0

        lax.fori_loop(0, L // 2, _row_body, 0)
        r0 = (b0 + b) * (L // 2)
        pltpu.sync_copy(trow.at[pl.ds(0, L // 2), pl.ds(0, D)],
                        out_hbm.at[pl.ds(r0, L // 2), pl.ds(0, D)])
        pltpu.sync_copy(trow.at[pl.ds(L // 2, L // 2), pl.ds(0, D)],
                        out_hbm.at[pl.ds(r0, L // 2), pl.ds(D, D)])

    _start(0, trow0_v, prow0_v, sem_t0, sem_p0)

    def _pipe_body(g, _):
        _start(2 * g + 1, trow1_v, prow1_v, sem_t1, sem_p1)
        _finish(2 * g, trow0_v, prow0_v, sem_t0, sem_p0)

        @pl.when(g < BPW // 2 - 1)
        def _():
            _start(2 * g + 2, trow0_v, prow0_v, sem_t0, sem_p0)

        _finish(2 * g + 1, trow1_v, prow1_v, sem_t1, sem_p1)
        return 0

    lax.fori_loop(0, BPW // 2, _pipe_body, 0)


def _sc_gather_sum(tok_table, ptab, tidx, pidx, sidx):
    mesh = plsc.VectorSubcoreMesh(core_axis_name="c", subcore_axis_name="s")
    fn = functools.partial(
        pl.kernel,
        mesh=mesh,
        compiler_params=pltpu.CompilerParams(use_tc_tiling_on_sc=False),
        out_type=jax.ShapeDtypeStruct((ROWS // 2, 2 * D), jnp.float32),
        scratch_types=[
            pltpu.VMEM((BPW, L), jnp.int32),
            pltpu.VMEM((BPW, L), jnp.int32),
            pltpu.VMEM((BPW, L), jnp.int32),
            pltpu.VMEM((BPW, L), jnp.int32),
            pltpu.VMEM((L, 2 * D), jnp.float32),
            pltpu.VMEM((L, D), jnp.float32),
            pltpu.VMEM((L, 2 * D), jnp.float32),
            pltpu.VMEM((L, D), jnp.float32),
            pltpu.SemaphoreType.DMA,
            pltpu.SemaphoreType.DMA,
            pltpu.SemaphoreType.DMA,
            pltpu.SemaphoreType.DMA,
        ],
    )(_sc_body)
    return fn(tok_table, ptab, tidx, pidx, sidx)


def _mask_body(m_ref, out_ref):
    out_ref[...] = jnp.broadcast_to(m_ref[...], (8, D, B))


def _build_mask(mask_row):
    out = pl.pallas_call(
        _mask_body,
        grid=(L // 8,),
        in_specs=[pl.BlockSpec((1, D, 1), lambda i: (0, 0, 0))],
        out_specs=pl.BlockSpec((8, D, B), lambda i: (i, 0, 0)),
        out_shape=jax.ShapeDtypeStruct((L, D, B), jnp.float32),
    )(mask_row)
    return out.transpose(2, 0, 1)


def _ln_body(x_ref, g_ref, b_ref, y_ref):
    x = x_ref[...].astype(jnp.float32)
    ii = lax.broadcasted_iota(jnp.int32, (2 * D, 2 * D), 0) // D
    jj = lax.broadcasted_iota(jnp.int32, (2 * D, 2 * D), 1) // D
    avg = jnp.where(ii == jj, 1.0 / D, 0.0).astype(jnp.float32)
    m = jnp.dot(x, avg, preferred_element_type=jnp.float32)
    s2 = jnp.dot(x * x, avg, preferred_element_type=jnp.float32)
    inv = lax.rsqrt(s2 - m * m + EPS)
    y = (x - m) * inv * g_ref[...] + b_ref[...]
    ya = y[:, :D].reshape(LN_B, L // 2, D)
    yb = y[:, D:].reshape(LN_B, L // 2, D)
    y_ref[...] = jnp.concatenate([ya, yb], axis=1)


def _ln_norm(packed, gamma2, beta2):
    grid = (B // LN_B,)
    return pl.pallas_call(
        _ln_body,
        grid=grid,
        in_specs=[
            pl.BlockSpec((PACK, 2 * D), lambda i: (i, 0)),
            pl.BlockSpec((1, 2 * D), lambda i: (0, 0)),
            pl.BlockSpec((1, 2 * D), lambda i: (0, 0)),
        ],
        out_specs=pl.BlockSpec((LN_B, L, D), lambda i: (i, 0, 0)),
        out_shape=jax.ShapeDtypeStruct((B, L, D), jnp.float32),
    )(packed, gamma2, beta2)


def kernel(input_token, position_ids, segment_ids, tok_table, type_table,
           pos_table, ln_gamma, ln_beta):
    ptab = _build_ptab(pos_table, type_table)
    tokp = _relayout_table(tok_table.T)
    packed = _sc_gather_sum(tokp, ptab, input_token, position_ids,
                            segment_ids)

    gamma2 = jnp.concatenate([ln_gamma, ln_gamma]).reshape(1, 2 * D)
    beta2 = jnp.concatenate([ln_beta, ln_beta]).reshape(1, 2 * D)
    mask_row = lax.slice(tok_table, (103, 0), (104, D))
    mask = _build_mask(mask_row.reshape(1, D, 1))
    y = _ln_norm(packed, gamma2, beta2)
    return y, mask
